# baseline (speedup 1.0000x reference)
import functools

import jax
import jax.numpy as jnp
from jax import lax
from jax.experimental import pallas as pl
from jax.experimental.pallas import tpu as pltpu
from jax.experimental.pallas import tpu_sc as plsc

N_TOK = 16384
N_EMB = 8192
DIM = 32
TILE = 256
GRID = N_TOK // TILE
COMMITMENT = 0.5


CHUNK = 4096
N_CHUNKS = N_EMB // CHUNK


def _argmin_body(xb2_ref, sx2_ref, se2_ref, emb_ref, idx_ref):
    xw = xb2_ref[...].astype(jnp.float32)
    sx2 = sx2_ref[...]
    acc = jnp.full((TILE, 1), jnp.inf, jnp.float32)
    idx = jnp.zeros((TILE, 1), jnp.int32)
    iota = lax.broadcasted_iota(jnp.int32, (TILE, CHUNK), 1)
    for c in range(N_CHUNKS):
        e_c = emb_ref[c * CHUNK:(c + 1) * CHUNK, :]
        se2_c = se2_ref[:, c * CHUNK:(c + 1) * CHUNK]
        mm2 = lax.dot_general(xw, e_c, (((1,), (1,)), ((), ())),
                              preferred_element_type=jnp.float32,
                              precision=lax.Precision.DEFAULT)
        d = (sx2 + se2_c) - mm2
        m = jnp.min(d, axis=1, keepdims=True)
        im = jnp.min(jnp.where(d == m, iota, 1 << 30), axis=1,
                     keepdims=True) + c * CHUNK
        take = m < acc
        idx = jnp.where(take, im, idx)
        acc = jnp.where(take, m.astype(jnp.bfloat16).astype(jnp.float32), acc)
    idx_ref[...] = idx


def _nearest_code_indices(xb, sx2, se2, embedding):
    return pl.pallas_call(
        _argmin_body,
        grid=(GRID,),
        in_specs=[
            pl.BlockSpec((TILE, DIM), lambda i: (i, 0)),
            pl.BlockSpec((TILE, 1), lambda i: (i, 0)),
            pl.BlockSpec((1, N_EMB), lambda i: (0, 0)),
            pl.BlockSpec((N_EMB, DIM), lambda i: (0, 0)),
        ],
        out_specs=pl.BlockSpec((TILE, 1), lambda i: (i, 0)),
        out_shape=jax.ShapeDtypeStruct((N_TOK, 1), jnp.int32),
    )(xb, sx2, se2, embedding)


@functools.lru_cache(maxsize=1)
def _make_sc_gather():
    info = plsc.get_sparse_core_info()
    nc, ns = info.num_cores, info.num_subcores
    nw = nc * ns
    bpw = N_TOK // nw
    mesh = plsc.VectorSubcoreMesh(core_axis_name="c", subcore_axis_name="s")

    @functools.partial(
        pl.kernel, mesh=mesh,
        out_type=[jax.ShapeDtypeStruct((N_TOK * DIM,), jnp.float32),
                  jax.ShapeDtypeStruct((nw * 16,), jnp.float32)],
        scratch_types=[
            pltpu.VMEM((bpw,), jnp.int32),
            pltpu.VMEM((bpw, 128), jnp.float32),
            pltpu.VMEM((bpw * DIM,), jnp.float32),
            pltpu.VMEM((bpw * DIM,), jnp.float32),
            pltpu.VMEM((16,), jnp.float32),
            pltpu.SemaphoreType.DMA,
        ],
    )
    def sc_gather(table_hbm, idx_hbm, x_hbm, qst_hbm, part_hbm,
                  idx_v, q_v, x_v, qst_v, acc_v, sem):
        wid = lax.axis_index("s") * nc + lax.axis_index("c")
        base = wid * bpw
        pltpu.sync_copy(idx_hbm.at[pl.ds(base, bpw)], idx_v)
        pltpu.async_copy(table_hbm.at[idx_v], q_v, sem).wait()
        pltpu.sync_copy(x_hbm.at[pl.ds(base * DIM, bpw * DIM)], x_v)

        def body(j, acc):
            for h in (0, 16):
                xv = x_v[pl.ds(j * DIM + h, 16)]
                qv = q_v[j, pl.ds(h, 16)]
                dv = qv - xv
                acc = acc + dv * dv
                qst_v[pl.ds(j * DIM + h, 16)] = xv + dv
            return acc

        acc = lax.fori_loop(0, bpw, body, jnp.zeros((16,), jnp.float32))
        acc_v[...] = acc
        pltpu.sync_copy(qst_v, qst_hbm.at[pl.ds(base * DIM, bpw * DIM)])
        pltpu.sync_copy(acc_v, part_hbm.at[pl.ds(wid * 16, 16)])

    return sc_gather


def kernel(inputs, embedding):
    flat = inputs.reshape(-1, DIM)
    sx2 = jnp.sum(flat ** 2, axis=1, keepdims=True)
    se2 = jnp.sum(embedding ** 2, axis=1)[None, :]
    xb2 = flat.astype(jnp.bfloat16) * jnp.bfloat16(2.0)
    idx = _nearest_code_indices(xb2, sx2, se2, embedding).reshape(-1)
    table_p = jnp.pad(embedding, ((0, 0), (0, 128 - DIM)))
    qst_flat, partials = _make_sc_gather()(table_p, idx, flat.reshape(-1))
    mse = jnp.sum(partials) / (N_TOK * DIM)
    loss = mse + COMMITMENT * mse
    return qst_flat.reshape(inputs.shape), loss, idx


# speedup vs baseline: 1.0533x; 1.0533x over previous
import functools

import jax
import jax.numpy as jnp
from jax import lax
from jax.experimental import pallas as pl
from jax.experimental.pallas import tpu as pltpu
from jax.experimental.pallas import tpu_sc as plsc

N_TOK = 16384
N_EMB = 8192
DIM = 32
TILE = 256
GRID = N_TOK // TILE
COMMITMENT = 0.5


CHUNK = 4096
N_CHUNKS = N_EMB // CHUNK


def _argmin_body(xb2_ref, sx2_ref, se2_ref, emb_ref, idx_ref):
    xw = xb2_ref[...].astype(jnp.float32)
    sx2 = sx2_ref[...]
    acc = jnp.full((TILE, 1), jnp.inf, jnp.float32)
    idx = jnp.zeros((TILE, 1), jnp.int32)
    iota = lax.broadcasted_iota(jnp.int32, (TILE, CHUNK), 1)
    for c in range(N_CHUNKS):
        e_c = emb_ref[c * CHUNK:(c + 1) * CHUNK, :]
        se2_c = se2_ref[:, c * CHUNK:(c + 1) * CHUNK]
        mm = lax.dot_general(xw, e_c, (((1,), (1,)), ((), ())),
                             preferred_element_type=jnp.float32,
                             precision=lax.Precision.DEFAULT)
        d = (sx2 + se2_c) - 2.0 * mm
        m = jnp.min(d, axis=1, keepdims=True)
        im = jnp.min(jnp.where(d == m, iota, 1 << 30), axis=1,
                     keepdims=True) + c * CHUNK
        take = m < acc
        idx = jnp.where(take, im, idx)
        acc = jnp.where(take, m.astype(jnp.bfloat16).astype(jnp.float32), acc)
    idx_ref[...] = idx


def _nearest_code_indices(xb, sx2, se2, embedding):
    return pl.pallas_call(
        _argmin_body,
        grid=(GRID,),
        in_specs=[
            pl.BlockSpec((TILE, DIM), lambda i: (i, 0)),
            pl.BlockSpec((TILE, 1), lambda i: (i, 0)),
            pl.BlockSpec((1, N_EMB), lambda i: (0, 0)),
            pl.BlockSpec((N_EMB, DIM), lambda i: (0, 0)),
        ],
        out_specs=pl.BlockSpec((TILE, 1), lambda i: (i, 0)),
        out_shape=jax.ShapeDtypeStruct((N_TOK, 1), jnp.int32),
    )(xb, sx2, se2, embedding)


@functools.lru_cache(maxsize=1)
def _make_sc_gather():
    info = plsc.get_sparse_core_info()
    nc, ns = info.num_cores, info.num_subcores
    nw = nc * ns
    bpw = N_TOK // nw
    mesh = plsc.VectorSubcoreMesh(core_axis_name="c", subcore_axis_name="s")

    @functools.partial(
        pl.kernel, mesh=mesh,
        out_type=[jax.ShapeDtypeStruct((N_TOK * DIM,), jnp.float32),
                  jax.ShapeDtypeStruct((nw * 16,), jnp.float32)],
        scratch_types=[
            pltpu.VMEM((bpw,), jnp.int32),
            pltpu.VMEM((bpw, 128), jnp.float32),
            pltpu.VMEM((bpw * DIM,), jnp.float32),
            pltpu.VMEM((bpw * DIM,), jnp.float32),
            pltpu.VMEM((16,), jnp.float32),
            pltpu.SemaphoreType.DMA,
        ],
    )
    def sc_gather(table_hbm, idx_hbm, x_hbm, qst_hbm, part_hbm,
                  idx_v, q_v, x_v, qst_v, acc_v, sem):
        wid = lax.axis_index("s") * nc + lax.axis_index("c")
        base = wid * bpw
        pltpu.sync_copy(idx_hbm.at[pl.ds(base, bpw)], idx_v)
        pltpu.async_copy(table_hbm.at[idx_v], q_v, sem).wait()
        pltpu.sync_copy(x_hbm.at[pl.ds(base * DIM, bpw * DIM)], x_v)

        def body(j, acc):
            for h in (0, 16):
                xv = x_v[pl.ds(j * DIM + h, 16)]
                qv = q_v[j, pl.ds(h, 16)]
                dv = qv - xv
                acc = acc + dv * dv
                qst_v[pl.ds(j * DIM + h, 16)] = xv + dv
            return acc

        acc = lax.fori_loop(0, bpw, body, jnp.zeros((16,), jnp.float32))
        acc_v[...] = acc
        pltpu.sync_copy(qst_v, qst_hbm.at[pl.ds(base * DIM, bpw * DIM)])
        pltpu.sync_copy(acc_v, part_hbm.at[pl.ds(wid * 16, 16)])

    return sc_gather


def kernel(inputs, embedding):
    flat = inputs.reshape(-1, DIM)
    sx2 = jnp.sum(flat ** 2, axis=1, keepdims=True)
    se2 = jnp.sum(embedding ** 2, axis=1)[None, :]
    xb = flat.astype(jnp.bfloat16)
    idx = _nearest_code_indices(xb, sx2, se2, embedding).reshape(-1)
    table_p = jnp.pad(embedding, ((0, 0), (0, 128 - DIM)))
    qst_flat, partials = _make_sc_gather()(table_p, idx, flat.reshape(-1))
    mse = jnp.sum(partials) / (N_TOK * DIM)
    loss = mse + COMMITMENT * mse
    return qst_flat.reshape(inputs.shape), loss, idx
